# baseline (speedup 1.0000x reference)
import jax
import jax.numpy as jnp
from jax import lax
from jax.experimental import pallas as pl
from jax.experimental.pallas import tpu as pltpu


def _dot(a, b):
    return lax.dot_general(
        a, b, (((1,), (0,)), ((), ())), preferred_element_type=jnp.float32
    )


def _dot_t(a, b):
    return lax.dot_general(
        a, b, (((1,), (1,)), ((), ())), preferred_element_type=jnp.float32
    )


def _full(shape):
    return pl.BlockSpec(shape, lambda i: (0,) * len(shape))


def _make_merged_body(nb, ar, n_seq, sr):
    def body(go_ref, seq_ref, adj_ref, mw1_ref, mb1_ref, mw2_ref, mb2_ref,
             gw1_ref, g1b_ref, gw2_ref, g2b_ref,
             f1w_ref, f1b_ref, f2w_ref, f2b_ref,
             hsem_ref, hstruct_ref, pred_ref,
             s1_ref, s2_ref, seqout16_ref):
        i = pl.program_id(0)

        @pl.when(i == 0)
        def _prologue():
            g = go_ref[...]
            h = jnp.maximum(_dot(g, mw1_ref[...]) + mb1_ref[...], 0.0)
            hsem = _dot(h, mw2_ref[...]) + mb2_ref[...]
            hsem_ref[...] = hsem
            s1_ref[...] = _dot(g, gw1_ref[...])

        @pl.when(i < n_seq)
        def _seq_prep():
            h = jnp.maximum(_dot(seq_ref[...], f1w_ref[...]) + f1b_ref[...],
                            0.0)
            so = _dot(h, f2w_ref[...]) + f2b_ref[...]
            seqout16_ref[pl.ds(i * sr, sr), :] = so.astype(jnp.bfloat16)

        @pl.when(i < nb)
        def _phase1():
            x = jnp.maximum(
                _dot(adj_ref[...], s1_ref[...]) + g1b_ref[...], 0.0)
            s2_ref[pl.ds(i * ar, ar), :] = _dot(x, gw2_ref[...])

        @pl.when(i >= nb)
        def _phase2():
            j = 2 * nb - 1 - i
            hs = jnp.maximum(
                _dot(adj_ref[...], s2_ref[...]) + g2b_ref[...], 0.0)
            hstruct_ref[...] = hs
            go_blk = jnp.concatenate(
                [hsem_ref[pl.ds(j * ar, ar), :].astype(jnp.bfloat16),
                 hs.astype(jnp.bfloat16)], axis=1)
            pred_ref[...] = jax.nn.sigmoid(_dot_t(seqout16_ref[...], go_blk))

    return body


def kernel(seq_embed, go_embed, adj, mlp_w1, mlp_b1, mlp_w2, mlp_b2,
           gc1_w, gc1_b, gc2_w, gc2_b, fc1_w, fc1_b, fc2_w, fc2_b):
    N, _ = adj.shape
    B, d_seq = seq_embed.shape
    go_feat = go_embed.shape[1]
    h0 = mlp_w1.shape[1]
    h1 = mlp_w2.shape[1]

    mb1 = mlp_b1.reshape(1, h0)
    mb2 = mlp_b2.reshape(1, h1)
    g1b = gc1_b.reshape(1, h0)
    g2b = gc2_b.reshape(1, h1)
    f1b = fc1_b.reshape(1, h0)
    f2b = fc2_b.reshape(1, 2 * h1)

    ar = min(256, N)
    sr = min(256, B)
    nb = N // ar
    n_seq = B // sr

    h_semantic, h_structure, pred = pl.pallas_call(
        _make_merged_body(nb, ar, n_seq, sr),
        grid=(2 * nb,),
        in_specs=[
            _full((N, go_feat)),
            pl.BlockSpec((sr, d_seq), lambda i: (lax.min(i, n_seq - 1), 0)),
            pl.BlockSpec(
                (ar, N),
                lambda i: (jnp.where(i < nb, i, 2 * nb - 1 - i), 0)),
            _full((go_feat, h0)), _full((1, h0)),
            _full((h0, h1)), _full((1, h1)),
            _full((go_feat, h0)), _full((1, h0)),
            _full((h0, h1)), _full((1, h1)),
            _full((d_seq, h0)), _full((1, h0)),
            _full((h0, 2 * h1)), _full((1, 2 * h1)),
        ],
        out_specs=[
            _full((N, h1)),
            pl.BlockSpec(
                (ar, h1),
                lambda i: (jnp.where(i < nb, nb - 1, 2 * nb - 1 - i), 0)),
            pl.BlockSpec(
                (B, ar),
                lambda i: (0, jnp.where(i < nb, nb - 1, 2 * nb - 1 - i))),
        ],
        out_shape=[
            jax.ShapeDtypeStruct((N, h1), jnp.float32),
            jax.ShapeDtypeStruct((N, h1), jnp.float32),
            jax.ShapeDtypeStruct((B, N), jnp.float32),
        ],
        scratch_shapes=[
            pltpu.VMEM((N, h0), jnp.float32),
            pltpu.VMEM((N, h1), jnp.float32),
            pltpu.VMEM((B, 2 * h1), jnp.bfloat16),
        ],
    )(go_embed, seq_embed, adj, mlp_w1, mb1, mlp_w2, mb2, gc1_w, g1b,
      gc2_w, g2b, fc1_w, f1b, fc2_w, f2b)

    return (h_semantic, h_structure, pred)


# speedup vs baseline: 1.0000x; 1.0000x over previous
import jax
import jax.numpy as jnp
from jax import lax
from jax.experimental import pallas as pl
from jax.experimental.pallas import tpu as pltpu


def _dot(a, b):
    return lax.dot_general(
        a, b, (((1,), (0,)), ((), ())), preferred_element_type=jnp.float32
    )


def _dot_t(a, b):
    return lax.dot_general(
        a, b, (((1,), (1,)), ((), ())), preferred_element_type=jnp.float32
    )


def _full(shape):
    return pl.BlockSpec(shape, lambda i: (0,) * len(shape))


def _make_merged_body(nb, ar, n_seq, sr):
    def body(go_ref, seq_ref, adj_ref, mw1_ref, mb1_ref, mw2_ref, mb2_ref,
             gw1_ref, g1b_ref, gw2_ref, g2b_ref,
             f1w_ref, f1b_ref, f2w_ref, f2b_ref,
             hsem_ref, hstruct_ref, pred_ref,
             s1_ref, s2_ref, seqout16_ref):
        i = pl.program_id(0)

        @pl.when(i == 0)
        def _prologue():
            g = go_ref[...]
            h = jnp.maximum(_dot(g, mw1_ref[...]) + mb1_ref[...], 0.0)
            hsem = _dot(h, mw2_ref[...]) + mb2_ref[...]
            hsem_ref[...] = hsem
            s1_ref[...] = _dot(g, gw1_ref[...])

        @pl.when(i < n_seq)
        def _seq_prep():
            h = jnp.maximum(_dot(seq_ref[...], f1w_ref[...]) + f1b_ref[...],
                            0.0)
            so = _dot(h, f2w_ref[...]) + f2b_ref[...]
            seqout16_ref[pl.ds(i * sr, sr), :] = so.astype(jnp.bfloat16)

        @pl.when(i < nb)
        def _phase1():
            x = jnp.maximum(
                _dot(adj_ref[...], s1_ref[...]) + g1b_ref[...], 0.0)
            s2_ref[pl.ds(i * ar, ar), :] = _dot(x, gw2_ref[...])

        @pl.when(i >= nb)
        def _phase2():
            j = 2 * nb - 1 - i
            hs = jnp.maximum(
                _dot(adj_ref[...], s2_ref[...]) + g2b_ref[...], 0.0)
            hstruct_ref[...] = hs
            go_blk = jnp.concatenate(
                [hsem_ref[pl.ds(j * ar, ar), :].astype(jnp.bfloat16),
                 hs.astype(jnp.bfloat16)], axis=1)
            logits = _dot_t(seqout16_ref[...], go_blk)
            pred_ref[...] = 0.5 * jnp.tanh(0.5 * logits) + 0.5

    return body


def kernel(seq_embed, go_embed, adj, mlp_w1, mlp_b1, mlp_w2, mlp_b2,
           gc1_w, gc1_b, gc2_w, gc2_b, fc1_w, fc1_b, fc2_w, fc2_b):
    N, _ = adj.shape
    B, d_seq = seq_embed.shape
    go_feat = go_embed.shape[1]
    h0 = mlp_w1.shape[1]
    h1 = mlp_w2.shape[1]

    mb1 = mlp_b1.reshape(1, h0)
    mb2 = mlp_b2.reshape(1, h1)
    g1b = gc1_b.reshape(1, h0)
    g2b = gc2_b.reshape(1, h1)
    f1b = fc1_b.reshape(1, h0)
    f2b = fc2_b.reshape(1, 2 * h1)

    ar = min(256, N)
    sr = min(256, B)
    nb = N // ar
    n_seq = B // sr

    h_semantic, h_structure, pred = pl.pallas_call(
        _make_merged_body(nb, ar, n_seq, sr),
        grid=(2 * nb,),
        in_specs=[
            _full((N, go_feat)),
            pl.BlockSpec((sr, d_seq), lambda i: (lax.min(i, n_seq - 1), 0)),
            pl.BlockSpec(
                (ar, N),
                lambda i: (jnp.where(i < nb, i, 2 * nb - 1 - i), 0)),
            _full((go_feat, h0)), _full((1, h0)),
            _full((h0, h1)), _full((1, h1)),
            _full((go_feat, h0)), _full((1, h0)),
            _full((h0, h1)), _full((1, h1)),
            _full((d_seq, h0)), _full((1, h0)),
            _full((h0, 2 * h1)), _full((1, 2 * h1)),
        ],
        out_specs=[
            _full((N, h1)),
            pl.BlockSpec(
                (ar, h1),
                lambda i: (jnp.where(i < nb, nb - 1, 2 * nb - 1 - i), 0)),
            pl.BlockSpec(
                (B, ar),
                lambda i: (0, jnp.where(i < nb, nb - 1, 2 * nb - 1 - i))),
        ],
        out_shape=[
            jax.ShapeDtypeStruct((N, h1), jnp.float32),
            jax.ShapeDtypeStruct((N, h1), jnp.float32),
            jax.ShapeDtypeStruct((B, N), jnp.float32),
        ],
        scratch_shapes=[
            pltpu.VMEM((N, h0), jnp.float32),
            pltpu.VMEM((N, h1), jnp.float32),
            pltpu.VMEM((B, 2 * h1), jnp.bfloat16),
        ],
    )(go_embed, seq_embed, adj, mlp_w1, mb1, mlp_w2, mb2, gc1_w, g1b,
      gc2_w, g2b, fc1_w, f1b, fc2_w, f2b)

    return (h_semantic, h_structure, pred)


# speedup vs baseline: 1.0276x; 1.0276x over previous
import jax
import jax.numpy as jnp
from jax import lax
from jax.experimental import pallas as pl
from jax.experimental.pallas import tpu as pltpu


def _dot(a, b):
    return lax.dot_general(
        a, b, (((1,), (0,)), ((), ())), preferred_element_type=jnp.float32
    )


def _dot_t(a, b):
    return lax.dot_general(
        a, b, (((1,), (1,)), ((), ())), preferred_element_type=jnp.float32
    )


def _full(shape):
    return pl.BlockSpec(shape, lambda i: (0,) * len(shape))


def _make_merged_body(nb, ar, n_seq, sr):
    def body(go_ref, seq_ref, adj_ref, mw1_ref, mb1_ref, mw2_ref, mb2_ref,
             gw1_ref, g1b_ref, gw2_ref, g2b_ref,
             f1w_ref, f1b_ref, f2w_ref, f2b_ref,
             hsem_ref, hstruct_ref, pred_ref,
             s1_ref, s2_ref, seqout16_ref, abuf_ref, sem_ref):
        i = pl.program_id(0)
        T = 2 * nb

        def _copy(k, slot):
            row = lax.rem(k, nb) * ar
            return pltpu.make_async_copy(
                adj_ref.at[pl.ds(row, ar), :],
                abuf_ref.at[slot],
                sem_ref.at[slot])

        @pl.when(i == 0)
        def _prime():
            _copy(0, 0).start()

        @pl.when(i + 1 < T)
        def _lookahead():
            _copy(i + 1, lax.rem(i + 1, 2)).start()

        @pl.when(i == 0)
        def _prologue():
            g = go_ref[...]
            h = jnp.maximum(_dot(g, mw1_ref[...]) + mb1_ref[...], 0.0)
            hsem = _dot(h, mw2_ref[...]) + mb2_ref[...]
            hsem_ref[...] = hsem
            s1_ref[...] = _dot(g, gw1_ref[...])

        @pl.when(i < n_seq)
        def _seq_prep():
            h = jnp.maximum(_dot(seq_ref[...], f1w_ref[...]) + f1b_ref[...],
                            0.0)
            so = _dot(h, f2w_ref[...]) + f2b_ref[...]
            seqout16_ref[pl.ds(i * sr, sr), :] = so.astype(jnp.bfloat16)

        _copy(i, lax.rem(i, 2)).wait()
        slot = lax.rem(i, 2)

        @pl.when(i < nb)
        def _phase1():
            x = jnp.maximum(
                _dot(abuf_ref[slot], s1_ref[...]) + g1b_ref[...], 0.0)
            s2_ref[pl.ds(i * ar, ar), :] = _dot(x, gw2_ref[...])

        @pl.when(i >= nb)
        def _phase2():
            j = i - nb
            hs = jnp.maximum(
                _dot(abuf_ref[slot], s2_ref[...]) + g2b_ref[...], 0.0)
            hstruct_ref[...] = hs
            go_blk = jnp.concatenate(
                [hsem_ref[pl.ds(j * ar, ar), :].astype(jnp.bfloat16),
                 hs.astype(jnp.bfloat16)], axis=1)
            logits = _dot_t(seqout16_ref[...], go_blk)
            pred_ref[...] = 0.5 * jnp.tanh(0.5 * logits) + 0.5

    return body


def kernel(seq_embed, go_embed, adj, mlp_w1, mlp_b1, mlp_w2, mlp_b2,
           gc1_w, gc1_b, gc2_w, gc2_b, fc1_w, fc1_b, fc2_w, fc2_b):
    N, _ = adj.shape
    B, d_seq = seq_embed.shape
    go_feat = go_embed.shape[1]
    h0 = mlp_w1.shape[1]
    h1 = mlp_w2.shape[1]

    mb1 = mlp_b1.reshape(1, h0)
    mb2 = mlp_b2.reshape(1, h1)
    g1b = gc1_b.reshape(1, h0)
    g2b = gc2_b.reshape(1, h1)
    f1b = fc1_b.reshape(1, h0)
    f2b = fc2_b.reshape(1, 2 * h1)

    ar = min(256, N)
    sr = min(256, B)
    nb = N // ar
    n_seq = B // sr

    h_semantic, h_structure, pred = pl.pallas_call(
        _make_merged_body(nb, ar, n_seq, sr),
        grid=(2 * nb,),
        in_specs=[
            _full((N, go_feat)),
            pl.BlockSpec((sr, d_seq), lambda i: (lax.min(i, n_seq - 1), 0)),
            pl.BlockSpec(memory_space=pl.ANY),
            _full((go_feat, h0)), _full((1, h0)),
            _full((h0, h1)), _full((1, h1)),
            _full((go_feat, h0)), _full((1, h0)),
            _full((h0, h1)), _full((1, h1)),
            _full((d_seq, h0)), _full((1, h0)),
            _full((h0, 2 * h1)), _full((1, 2 * h1)),
        ],
        out_specs=[
            _full((N, h1)),
            pl.BlockSpec((ar, h1), lambda i: (lax.max(i - nb, 0), 0)),
            pl.BlockSpec((B, ar), lambda i: (0, lax.max(i - nb, 0))),
        ],
        out_shape=[
            jax.ShapeDtypeStruct((N, h1), jnp.float32),
            jax.ShapeDtypeStruct((N, h1), jnp.float32),
            jax.ShapeDtypeStruct((B, N), jnp.float32),
        ],
        scratch_shapes=[
            pltpu.VMEM((N, h0), jnp.float32),
            pltpu.VMEM((N, h1), jnp.float32),
            pltpu.VMEM((B, 2 * h1), jnp.bfloat16),
            pltpu.VMEM((2, ar, N), jnp.float32),
            pltpu.SemaphoreType.DMA((2,)),
        ],
    )(go_embed, seq_embed, adj, mlp_w1, mb1, mlp_w2, mb2, gc1_w, g1b,
      gc2_w, g2b, fc1_w, f1b, fc2_w, f2b)

    return (h_semantic, h_structure, pred)
